# Initial kernel scaffold; baseline (speedup 1.0000x reference)
import jax
import jax.numpy as jnp
from jax.experimental import pallas as pl


def kernel(token_ids, node_emb, edge_weights, edge_matrix, eta, W, b):
    raise NotImplementedError("write your pallas kernel here")


# speedup vs baseline: 47.2516x; 47.2516x over previous
import functools

import jax
import jax.numpy as jnp
from jax import lax
from jax.experimental import pallas as pl
from jax.experimental.pallas import tpu as pltpu
from jax.experimental.pallas import tpu_sc as plsc

_B, _T, _V, _D = 1024, 50, 1000, 768
_E = _V * _V

_NW = 32
_CHUNK = 128
_ROWS_PER_W = 248
_EPAD = _NW * _ROWS_PER_W * _CHUNK

_TP = 56
_NB = 8
_NBT = _NB * _TP
_SCH = 8
_NEG = -1e30
_PREC = lax.Precision.HIGHEST


def _sc_edge_gather(em_rows, ew_flat):
    mesh = plsc.VectorSubcoreMesh(core_axis_name="c", subcore_axis_name="s")

    @functools.partial(
        pl.kernel,
        mesh=mesh,
        out_type=jax.ShapeDtypeStruct((_NW * _ROWS_PER_W, _CHUNK), jnp.float32),
        scratch_types=[
            pltpu.VMEM((_ROWS_PER_W, _CHUNK), jnp.int32),
            pltpu.VMEM((_ROWS_PER_W, _CHUNK), jnp.float32),
            pltpu.SemaphoreType.DMA,
        ],
    )
    def k(em_hbm, ew_hbm, out_hbm, idx_v, val_v, sem):
        wid = lax.axis_index("s") * 2 + lax.axis_index("c")
        r0 = wid * _ROWS_PER_W
        pltpu.sync_copy(em_hbm.at[pl.ds(r0, _ROWS_PER_W)], idx_v)

        look = 8

        def start(j):
            pltpu.async_copy(ew_hbm.at[idx_v.at[j]], val_v.at[j], sem)

        def drain(j):
            pltpu.make_async_copy(ew_hbm.at[idx_v.at[j]], val_v.at[j], sem).wait()

        for j in range(look):
            start(j)

        def body(j, carry):
            start(j)
            drain(j - look)
            return carry

        lax.fori_loop(look, _ROWS_PER_W, body, 0)

        def tail(j, carry):
            drain(j)
            return carry

        lax.fori_loop(_ROWS_PER_W - look, _ROWS_PER_W, tail, 0)
        pltpu.sync_copy(val_v, out_hbm.at[pl.ds(r0, _ROWS_PER_W)])

    return k(em_rows, ew_flat)


def _tc_body(tok_ref, ne_ref, ew_ref, eta_ref, wt_ref, b_ref, out_ref):
    tokc = tok_ref[...]
    etav = eta_ref[...]

    oh = (tokc == lax.broadcasted_iota(jnp.int32, (_NBT, _V), 1)).astype(
        jnp.float32)
    h_all = jnp.dot(oh, ne_ref[...], precision=_PREC,
                    preferred_element_type=jnp.float32)
    ewr_all = jnp.dot(oh, ew_ref[...], precision=_PREC,
                      preferred_element_type=jnp.float32)

    pos_r = lax.broadcasted_iota(jnp.int32, (_TP, _TP), 0)
    pos_c = lax.broadcasted_iota(jnp.int32, (_TP, _TP), 1)

    ge_rows = []
    for i in range(_NB):
        sl = slice(i * _TP, (i + 1) * _TP)
        tcol = tokc[sl, :]
        ohi = oh[sl, :]
        hi = h_all[sl, :]
        ewr = ewr_all[sl, :]
        wmat = lax.dot_general(ewr, ohi, (((1,), (1,)), ((), ())),
                               precision=_PREC,
                               preferred_element_type=jnp.float32)
        validc = tcol != 0
        hm = jnp.where(validc, hi, _NEG)
        wm = jnp.where(validc, wmat, 1.0)

        acc = jnp.full((_TP, _D), _NEG, jnp.float32)
        for c in range(_TP // _SCH):
            ssl = slice(c * _SCH, (c + 1) * _SCH)
            msg = hm[ssl, :][:, None, :] * wm[ssl, :][:, :, None]
            acc = jnp.maximum(acc, jnp.max(msg, axis=0))

        eqm = lax.dot_general(ohi, ohi, (((1,), (1,)), ((), ())),
                              precision=_PREC,
                              preferred_element_type=jnp.float32)
        dupsum = jnp.sum(jnp.where(pos_c < pos_r, eqm, 0.0), axis=1,
                         keepdims=True)
        nm = jnp.where(validc & (dupsum < 0.5), 1.0, 0.0)
        cnt = jnp.maximum(jnp.sum(nm), 1.0)
        new_h = etav * hi + (1.0 - etav) * acc
        gsum = jnp.sum(new_h * nm, axis=0, keepdims=True)
        ge_rows.append(gsum / cnt)

    ge = jnp.concatenate(ge_rows, axis=0)
    logits = jnp.dot(ge, wt_ref[...], precision=_PREC,
                     preferred_element_type=jnp.float32) + b_ref[...]
    out_ref[...] = jax.nn.sigmoid(logits)


def _tc_forward(tok_col, node_emb, ew_tab, eta2, wt, b2):
    return pl.pallas_call(
        _tc_body,
        grid=(_B // _NB,),
        in_specs=[
            pl.BlockSpec((_NBT, 1), lambda i: (i, 0)),
            pl.BlockSpec((_V, _D), lambda i: (0, 0)),
            pl.BlockSpec((_V, _V), lambda i: (0, 0)),
            pl.BlockSpec((1, 1), lambda i: (0, 0)),
            pl.BlockSpec((_D, _D), lambda i: (0, 0)),
            pl.BlockSpec((1, _D), lambda i: (0, 0)),
        ],
        out_specs=pl.BlockSpec((_NB, _D), lambda i: (i, 0)),
        out_shape=jax.ShapeDtypeStruct((_B, _D), jnp.float32),
        compiler_params=pltpu.CompilerParams(
            dimension_semantics=("arbitrary",)),
    )(tok_col, node_emb, ew_tab, eta2, wt, b2)


def kernel(token_ids, node_emb, edge_weights, edge_matrix, eta, W, b):
    em_flat = edge_matrix.reshape(-1)
    em_pad = jnp.concatenate(
        [em_flat, jnp.zeros((_EPAD - _E,), jnp.int32)])
    em_rows = em_pad.reshape(_NW * _ROWS_PER_W, _CHUNK)
    ew_flat = edge_weights.reshape(-1)
    gathered = _sc_edge_gather(em_rows, ew_flat)
    ew_tab = gathered.reshape(-1)[:_E].reshape(_V, _V)

    tok_pad = jnp.pad(token_ids, ((0, 0), (0, _TP - _T)))
    tok_col = tok_pad.reshape(_B * _TP, 1)
    return _tc_forward(tok_col, node_emb, ew_tab, eta.reshape(1, 1),
                       W.T, b.reshape(1, _D))


# speedup vs baseline: 75.9305x; 1.6069x over previous
import functools

import jax
import jax.numpy as jnp
from jax import lax
from jax.experimental import pallas as pl
from jax.experimental.pallas import tpu as pltpu
from jax.experimental.pallas import tpu_sc as plsc

_B, _T, _V, _D = 1024, 50, 1000, 768
_E = _V * _V

_NW = 32
_CHUNK = 128
_ROWS_PER_W = 248
_EPAD = _NW * _ROWS_PER_W * _CHUNK

_TP = 56
_NB = 8
_NBT = _NB * _TP
_SCH = 8
_NEG = -1e30
_PREC = lax.Precision.HIGHEST


def _sc_edge_gather(em_rows, ew_flat):
    mesh = plsc.VectorSubcoreMesh(core_axis_name="c", subcore_axis_name="s")

    @functools.partial(
        pl.kernel,
        mesh=mesh,
        out_type=jax.ShapeDtypeStruct((_NW * _ROWS_PER_W, _CHUNK), jnp.float32),
        scratch_types=[
            pltpu.VMEM((_ROWS_PER_W, _CHUNK), jnp.int32),
            pltpu.VMEM((_ROWS_PER_W, _CHUNK), jnp.float32),
            pltpu.SemaphoreType.DMA,
        ],
    )
    def k(em_hbm, ew_hbm, out_hbm, idx_v, val_v, sem):
        wid = lax.axis_index("s") * 2 + lax.axis_index("c")
        r0 = wid * _ROWS_PER_W
        pltpu.sync_copy(em_hbm.at[pl.ds(r0, _ROWS_PER_W)], idx_v)

        look = 8

        def start(j):
            pltpu.async_copy(ew_hbm.at[idx_v.at[j]], val_v.at[j], sem)

        def drain(j):
            pltpu.make_async_copy(ew_hbm.at[idx_v.at[j]], val_v.at[j], sem).wait()

        for j in range(look):
            start(j)

        def body(j, carry):
            start(j)
            drain(j - look)
            return carry

        lax.fori_loop(look, _ROWS_PER_W, body, 0)

        def tail(j, carry):
            drain(j)
            return carry

        lax.fori_loop(_ROWS_PER_W - look, _ROWS_PER_W, tail, 0)
        pltpu.sync_copy(val_v, out_hbm.at[pl.ds(r0, _ROWS_PER_W)])

    return k(em_rows, ew_flat)


def _tc_body(tok_ref, neh_ref, nel_ref, ewh_ref, ewl_ref, eta_ref, wt_ref,
             b_ref, out_ref):
    tokc = tok_ref[...]
    etav = eta_ref[...]

    oh = (tokc == lax.broadcasted_iota(jnp.int32, (_NBT, _V), 1)).astype(
        jnp.bfloat16)
    h_all = (jnp.dot(oh, neh_ref[...], preferred_element_type=jnp.float32)
             + jnp.dot(oh, nel_ref[...], preferred_element_type=jnp.float32))
    ewr_h = jnp.dot(oh, ewh_ref[...],
                    preferred_element_type=jnp.float32).astype(jnp.bfloat16)
    ewr_l = jnp.dot(oh, ewl_ref[...],
                    preferred_element_type=jnp.float32).astype(jnp.bfloat16)

    pos_r = lax.broadcasted_iota(jnp.int32, (_TP, _TP), 0)
    pos_c = lax.broadcasted_iota(jnp.int32, (_TP, _TP), 1)

    ge_rows = []
    for i in range(_NB):
        sl = slice(i * _TP, (i + 1) * _TP)
        tcol = tokc[sl, :]
        ohi = oh[sl, :]
        hi = h_all[sl, :]
        wmat = (lax.dot_general(ewr_h[sl, :], ohi, (((1,), (1,)), ((), ())),
                                preferred_element_type=jnp.float32)
                + lax.dot_general(ewr_l[sl, :], ohi, (((1,), (1,)), ((), ())),
                                  preferred_element_type=jnp.float32))
        validc = tcol != 0
        hm = jnp.where(validc, hi, _NEG)
        wm = jnp.where(validc, wmat, 1.0)

        acc = jnp.full((_TP, _D), _NEG, jnp.float32)
        for c in range(_TP // _SCH):
            ssl = slice(c * _SCH, (c + 1) * _SCH)
            msg = hm[ssl, :][:, None, :] * wm[ssl, :][:, :, None]
            acc = jnp.maximum(acc, jnp.max(msg, axis=0))

        eqm = lax.dot_general(ohi, ohi, (((1,), (1,)), ((), ())),
                              preferred_element_type=jnp.float32)
        dupsum = jnp.sum(jnp.where(pos_c < pos_r, eqm, 0.0), axis=1,
                         keepdims=True)
        nm = jnp.where(validc & (dupsum < 0.5), 1.0, 0.0)
        cnt = jnp.maximum(jnp.sum(nm), 1.0)
        new_h = etav * hi + (1.0 - etav) * acc
        gsum = jnp.sum(new_h * nm, axis=0, keepdims=True)
        ge_rows.append(gsum / cnt)

    ge = jnp.concatenate(ge_rows, axis=0)
    logits = jnp.dot(ge, wt_ref[...], precision=_PREC,
                     preferred_element_type=jnp.float32) + b_ref[...]
    out_ref[...] = jax.nn.sigmoid(logits)


def _split_bf16(x):
    hi = x.astype(jnp.bfloat16)
    lo = (x - hi.astype(jnp.float32)).astype(jnp.bfloat16)
    return hi, lo


def _tc_forward(tok_col, ne_hi, ne_lo, ew_hi, ew_lo, eta2, wt, b2):
    return pl.pallas_call(
        _tc_body,
        grid=(_B // _NB,),
        in_specs=[
            pl.BlockSpec((_NBT, 1), lambda i: (i, 0)),
            pl.BlockSpec((_V, _D), lambda i: (0, 0)),
            pl.BlockSpec((_V, _D), lambda i: (0, 0)),
            pl.BlockSpec((_V, _V), lambda i: (0, 0)),
            pl.BlockSpec((_V, _V), lambda i: (0, 0)),
            pl.BlockSpec((1, 1), lambda i: (0, 0)),
            pl.BlockSpec((_D, _D), lambda i: (0, 0)),
            pl.BlockSpec((1, _D), lambda i: (0, 0)),
        ],
        out_specs=pl.BlockSpec((_NB, _D), lambda i: (i, 0)),
        out_shape=jax.ShapeDtypeStruct((_B, _D), jnp.float32),
        compiler_params=pltpu.CompilerParams(
            dimension_semantics=("arbitrary",)),
    )(tok_col, ne_hi, ne_lo, ew_hi, ew_lo, eta2, wt, b2)


def kernel(token_ids, node_emb, edge_weights, edge_matrix, eta, W, b):
    em_flat = edge_matrix.reshape(-1)
    em_pad = jnp.concatenate(
        [em_flat, jnp.zeros((_EPAD - _E,), jnp.int32)])
    em_rows = em_pad.reshape(_NW * _ROWS_PER_W, _CHUNK)
    ew_flat = edge_weights.reshape(-1)
    gathered = _sc_edge_gather(em_rows, ew_flat)
    ew_tab = gathered.reshape(-1)[:_E].reshape(_V, _V)

    tok_pad = jnp.pad(token_ids, ((0, 0), (0, _TP - _T)))
    tok_col = tok_pad.reshape(_B * _TP, 1)
    ne_hi, ne_lo = _split_bf16(node_emb)
    ew_hi, ew_lo = _split_bf16(ew_tab)
    return _tc_forward(tok_col, ne_hi, ne_lo, ew_hi, ew_lo, eta.reshape(1, 1),
                       W.T, b.reshape(1, _D))


# speedup vs baseline: 134.7265x; 1.7743x over previous
import functools

import jax
import jax.numpy as jnp
from jax import lax
from jax.experimental import pallas as pl
from jax.experimental.pallas import tpu as pltpu
from jax.experimental.pallas import tpu_sc as plsc

_B, _T, _V, _D = 1024, 50, 1000, 768
_E = _V * _V

_NW = 32
_CHUNK = 128
_ROWS_PER_W = 248
_EPAD = _NW * _ROWS_PER_W * _CHUNK

_TP = 56
_NB = 8
_NBT = _NB * _TP
_DCH = 384
_NEG = -1e30
_PREC = lax.Precision.HIGHEST


def _sc_edge_gather(em_rows, ew_flat):
    mesh = plsc.VectorSubcoreMesh(core_axis_name="c", subcore_axis_name="s")

    @functools.partial(
        pl.kernel,
        mesh=mesh,
        out_type=jax.ShapeDtypeStruct((_NW * _ROWS_PER_W, _CHUNK), jnp.float32),
        scratch_types=[
            pltpu.VMEM((_ROWS_PER_W, _CHUNK), jnp.int32),
            pltpu.VMEM((_ROWS_PER_W, _CHUNK), jnp.float32),
            pltpu.SemaphoreType.DMA,
        ],
    )
    def k(em_hbm, ew_hbm, out_hbm, idx_v, val_v, sem):
        wid = lax.axis_index("s") * 2 + lax.axis_index("c")
        r0 = wid * _ROWS_PER_W
        pltpu.sync_copy(em_hbm.at[pl.ds(r0, _ROWS_PER_W)], idx_v)

        look = 8

        def start(j):
            pltpu.async_copy(ew_hbm.at[idx_v.at[j]], val_v.at[j], sem)

        def drain(j):
            pltpu.make_async_copy(ew_hbm.at[idx_v.at[j]], val_v.at[j], sem).wait()

        for j in range(look):
            start(j)

        def body(j, carry):
            start(j)
            drain(j - look)
            return carry

        lax.fori_loop(look, _ROWS_PER_W, body, 0)

        def tail(j, carry):
            drain(j)
            return carry

        lax.fori_loop(_ROWS_PER_W - look, _ROWS_PER_W, tail, 0)
        pltpu.sync_copy(val_v, out_hbm.at[pl.ds(r0, _ROWS_PER_W)])

    return k(em_rows, ew_flat)


def _tc_body(tok_ref, tokr_ref, neh_ref, nel_ref, ewh_ref, eta_ref,
             out_ref):
    tokc = tok_ref[...]
    tokr = tokr_ref[...]
    etav = eta_ref[...]

    oh = (tokc == lax.broadcasted_iota(jnp.int32, (_NBT, _V), 1)).astype(
        jnp.bfloat16)
    h_all = (jnp.dot(oh, neh_ref[...], preferred_element_type=jnp.float32)
             + jnp.dot(oh, nel_ref[...], preferred_element_type=jnp.float32))
    ewr_h = jnp.dot(oh, ewh_ref[...],
                    preferred_element_type=jnp.float32).astype(jnp.bfloat16)

    pos_r = lax.broadcasted_iota(jnp.int32, (_TP, _TP), 0)
    pos_c = lax.broadcasted_iota(jnp.int32, (_TP, _TP), 1)

    ge_rows = []
    for i in range(_NB):
        sl = slice(i * _TP, (i + 1) * _TP)
        tcol = tokc[sl, :]
        trow = tokr[i:i + 1, :]
        ohi = oh[sl, :]
        hi = h_all[sl, :]
        wmT = lax.dot_general(ohi, ewr_h[sl, :], (((1,), (1,)), ((), ())),
                              preferred_element_type=jnp.float32
                              ).astype(jnp.bfloat16)
        validc = tcol != 0
        hm = jnp.where(validc, hi, _NEG).astype(jnp.bfloat16)
        wmTm = jnp.where(trow != 0, wmT, jnp.bfloat16(1.0)).astype(
            jnp.bfloat16)

        acc_chunks = []
        for dc in range(0, _D, _DCH):
            dsl = slice(dc, dc + _DCH)
            a = wmTm[:, 0:1] * hm[0:1, dsl]
            for s in range(1, _T):
                a = jnp.maximum(a, wmTm[:, s:s + 1] * hm[s:s + 1, dsl])
            acc_chunks.append(a)
        acc = jnp.concatenate(acc_chunks, axis=1).astype(jnp.float32)

        dup = jnp.any((tcol == trow) & (pos_c < pos_r), axis=1,
                      keepdims=True)
        nm = jnp.where(validc & (~dup), 1.0, 0.0)
        cnt = jnp.maximum(jnp.sum(nm), 1.0)
        new_h = etav * hi + (1.0 - etav) * acc
        gsum = jnp.sum(new_h * nm, axis=0, keepdims=True)
        ge_rows.append(gsum / cnt)

    out_ref[...] = jnp.concatenate(ge_rows, axis=0)


def _fc_body(ge_ref, wth_ref, wtl_ref, b_ref, out_ref):
    ge = ge_ref[...]
    geh = ge.astype(jnp.bfloat16)
    gel = (ge - geh.astype(jnp.float32)).astype(jnp.bfloat16)
    logits = (jnp.dot(geh, wth_ref[...], preferred_element_type=jnp.float32)
              + jnp.dot(geh, wtl_ref[...], preferred_element_type=jnp.float32)
              + jnp.dot(gel, wth_ref[...], preferred_element_type=jnp.float32)
              + b_ref[...])
    out_ref[...] = jax.nn.sigmoid(logits)


def _split_bf16(x):
    hi = x.astype(jnp.bfloat16)
    lo = (x - hi.astype(jnp.float32)).astype(jnp.bfloat16)
    return hi, lo


def _tc_forward(tok_col, tok_pad, ne_hi, ne_lo, ew_hi, eta2):
    return pl.pallas_call(
        _tc_body,
        grid=(_B // _NB,),
        in_specs=[
            pl.BlockSpec((_NBT, 1), lambda i: (i, 0)),
            pl.BlockSpec((_NB, _TP), lambda i: (i, 0)),
            pl.BlockSpec((_V, _D), lambda i: (0, 0)),
            pl.BlockSpec((_V, _D), lambda i: (0, 0)),
            pl.BlockSpec((_V, _V), lambda i: (0, 0)),
            pl.BlockSpec((1, 1), lambda i: (0, 0)),
        ],
        out_specs=pl.BlockSpec((_NB, _D), lambda i: (i, 0)),
        out_shape=jax.ShapeDtypeStruct((_B, _D), jnp.float32),
        compiler_params=pltpu.CompilerParams(
            dimension_semantics=("arbitrary",)),
    )(tok_col, tok_pad, ne_hi, ne_lo, ew_hi, eta2)


_FCB = 256


def _fc_forward(ge, wt_hi, wt_lo, b2):
    return pl.pallas_call(
        _fc_body,
        grid=(_B // _FCB,),
        in_specs=[
            pl.BlockSpec((_FCB, _D), lambda i: (i, 0)),
            pl.BlockSpec((_D, _D), lambda i: (0, 0)),
            pl.BlockSpec((_D, _D), lambda i: (0, 0)),
            pl.BlockSpec((1, _D), lambda i: (0, 0)),
        ],
        out_specs=pl.BlockSpec((_FCB, _D), lambda i: (i, 0)),
        out_shape=jax.ShapeDtypeStruct((_B, _D), jnp.float32),
        compiler_params=pltpu.CompilerParams(
            dimension_semantics=("arbitrary",)),
    )(ge, wt_hi, wt_lo, b2)


def kernel(token_ids, node_emb, edge_weights, edge_matrix, eta, W, b):
    em_flat = edge_matrix.reshape(-1)
    em_pad = jnp.concatenate(
        [em_flat, jnp.zeros((_EPAD - _E,), jnp.int32)])
    em_rows = em_pad.reshape(_NW * _ROWS_PER_W, _CHUNK)
    ew_flat = edge_weights.reshape(-1)
    gathered = _sc_edge_gather(em_rows, ew_flat)
    ew_tab = gathered.reshape(-1)[:_E].reshape(_V, _V)

    tok_pad = jnp.pad(token_ids, ((0, 0), (0, _TP - _T)))
    tok_col = tok_pad.reshape(_B * _TP, 1)
    ne_hi, ne_lo = _split_bf16(node_emb)
    ew_hi = ew_tab.astype(jnp.bfloat16)
    ge = _tc_forward(tok_col, tok_pad, ne_hi, ne_lo, ew_hi,
                     eta.reshape(1, 1))
    wt_hi, wt_lo = _split_bf16(W.T)
    return _fc_forward(ge, wt_hi, wt_lo, b.reshape(1, _D))


# speedup vs baseline: 160.5099x; 1.1914x over previous
import functools

import jax
import jax.numpy as jnp
from jax import lax
from jax.experimental import pallas as pl
from jax.experimental.pallas import tpu as pltpu
from jax.experimental.pallas import tpu_sc as plsc

_B, _T, _V, _D = 1024, 50, 1000, 768
_E = _V * _V

_NW = 32
_CHUNK = 128
_ROWS_PER_W = 248
_EPAD = _NW * _ROWS_PER_W * _CHUNK

_TP = 56
_NB = 8
_NBT = _NB * _TP
_DCH = 384
_SCK = 2
_NEG = -1e30
_PREC = lax.Precision.HIGHEST


def _sc_edge_gather(em_rows, ew_flat):
    mesh = plsc.VectorSubcoreMesh(core_axis_name="c", subcore_axis_name="s")

    @functools.partial(
        pl.kernel,
        mesh=mesh,
        out_type=jax.ShapeDtypeStruct((_NW * _ROWS_PER_W, _CHUNK), jnp.float32),
        scratch_types=[
            pltpu.VMEM((_ROWS_PER_W, _CHUNK), jnp.int32),
            pltpu.VMEM((_ROWS_PER_W, _CHUNK), jnp.float32),
            pltpu.SemaphoreType.DMA,
        ],
    )
    def k(em_hbm, ew_hbm, out_hbm, idx_v, val_v, sem):
        wid = lax.axis_index("s") * 2 + lax.axis_index("c")
        r0 = wid * _ROWS_PER_W
        pltpu.sync_copy(em_hbm.at[pl.ds(r0, _ROWS_PER_W)], idx_v)

        look = 8

        def start(j):
            pltpu.async_copy(ew_hbm.at[idx_v.at[j]], val_v.at[j], sem)

        def drain(j):
            pltpu.make_async_copy(ew_hbm.at[idx_v.at[j]], val_v.at[j], sem).wait()

        for j in range(look):
            start(j)

        def body(j, carry):
            start(j)
            drain(j - look)
            return carry

        lax.fori_loop(look, _ROWS_PER_W, body, 0)

        def tail(j, carry):
            drain(j)
            return carry

        lax.fori_loop(_ROWS_PER_W - look, _ROWS_PER_W, tail, 0)
        pltpu.sync_copy(val_v, out_hbm.at[pl.ds(r0, _ROWS_PER_W)])

    return k(em_rows, ew_flat)


def _tc_body(tok_ref, tokr_ref, neh_ref, nel_ref, ewh_ref, eta_ref,
             out_ref):
    tokc = tok_ref[...]
    tokr = tokr_ref[...]
    etav = eta_ref[...]

    pos_r = lax.broadcasted_iota(jnp.int32, (_TP, _TP), 0)
    pos_c = lax.broadcasted_iota(jnp.int32, (_TP, _TP), 1)
    neg_bf = jnp.bfloat16(_NEG)

    u_rows, asum_rows, cnt_rows = [], [], []
    for g in range(0, _NB, _SCK):
        gsl = slice(g * _TP, (g + _SCK) * _TP)
        tg = tokc[gsl, :]
        ohg = (tg == lax.broadcasted_iota(
            jnp.int32, (_SCK * _TP, _V), 1)).astype(jnp.bfloat16)
        hg = jnp.dot(ohg, neh_ref[...],
                     preferred_element_type=jnp.float32).astype(jnp.bfloat16)
        ewrg = jnp.dot(ohg, ewh_ref[...],
                       preferred_element_type=jnp.float32).astype(jnp.bfloat16)
        for k in range(_SCK):
            i = g + k
            sl = slice(k * _TP, (k + 1) * _TP)
            tcol = tg[sl, :]
            trow = tokr[i:i + 1, :]
            ohi = ohg[sl, :]
            wmT = lax.dot_general(ohi, ewrg[sl, :], (((1,), (1,)), ((), ())),
                                  preferred_element_type=jnp.float32
                                  ).astype(jnp.bfloat16)
            validc = tcol != 0
            hm = jnp.where(validc, hg[sl, :], neg_bf)
            wmTm = jnp.where(trow != 0, wmT, jnp.bfloat16(1.0))

            acc_chunks = []
            for dc in range(0, _D, _DCH):
                dsl = slice(dc, dc + _DCH)
                a = wmTm[:, 0:1] * hm[0:1, dsl]
                for s in range(1, _T):
                    a = jnp.maximum(a, wmTm[:, s:s + 1] * hm[s:s + 1, dsl])
                acc_chunks.append(a)
            acc = jnp.concatenate(acc_chunks, axis=1).astype(jnp.float32)

            dup = jnp.any((tcol == trow) & (pos_c < pos_r), axis=1,
                          keepdims=True)
            nm = jnp.where(validc & (~dup), 1.0, 0.0)
            cnt_rows.append(jnp.maximum(jnp.sum(nm), 1.0).reshape(1, 1))
            asum_rows.append(jnp.sum(acc * nm, axis=0, keepdims=True))
            u_rows.append(lax.dot_general(
                nm.astype(jnp.bfloat16), ohi, (((0,), (0,)), ((), ())),
                preferred_element_type=jnp.float32).astype(jnp.bfloat16))

    u_mat = jnp.concatenate(u_rows, axis=0)
    hsum = (jnp.dot(u_mat, neh_ref[...], preferred_element_type=jnp.float32)
            + jnp.dot(u_mat, nel_ref[...], preferred_element_type=jnp.float32))
    asum = jnp.concatenate(asum_rows, axis=0)
    cnts = jnp.concatenate(cnt_rows, axis=0)
    out_ref[...] = (etav * hsum + (1.0 - etav) * asum) / cnts


def _fc_body(ge_ref, wth_ref, wtl_ref, b_ref, out_ref):
    ge = ge_ref[...]
    geh = ge.astype(jnp.bfloat16)
    gel = (ge - geh.astype(jnp.float32)).astype(jnp.bfloat16)
    logits = (jnp.dot(geh, wth_ref[...], preferred_element_type=jnp.float32)
              + jnp.dot(geh, wtl_ref[...], preferred_element_type=jnp.float32)
              + jnp.dot(gel, wth_ref[...], preferred_element_type=jnp.float32)
              + b_ref[...])
    out_ref[...] = jax.nn.sigmoid(logits)


def _split_bf16(x):
    hi = x.astype(jnp.bfloat16)
    lo = (x - hi.astype(jnp.float32)).astype(jnp.bfloat16)
    return hi, lo


def _tc_forward(tok_col, tok_pad, ne_hi, ne_lo, ew_hi, eta2):
    return pl.pallas_call(
        _tc_body,
        grid=(_B // _NB,),
        in_specs=[
            pl.BlockSpec((_NBT, 1), lambda i: (i, 0)),
            pl.BlockSpec((_NB, _TP), lambda i: (i, 0)),
            pl.BlockSpec((_V, _D), lambda i: (0, 0)),
            pl.BlockSpec((_V, _D), lambda i: (0, 0)),
            pl.BlockSpec((_V, _V), lambda i: (0, 0)),
            pl.BlockSpec((1, 1), lambda i: (0, 0)),
        ],
        out_specs=pl.BlockSpec((_NB, _D), lambda i: (i, 0)),
        out_shape=jax.ShapeDtypeStruct((_B, _D), jnp.float32),
        compiler_params=pltpu.CompilerParams(
            dimension_semantics=("parallel",)),
    )(tok_col, tok_pad, ne_hi, ne_lo, ew_hi, eta2)


_FCB = 256


def _fc_forward(ge, wt_hi, wt_lo, b2):
    return pl.pallas_call(
        _fc_body,
        grid=(_B // _FCB,),
        in_specs=[
            pl.BlockSpec((_FCB, _D), lambda i: (i, 0)),
            pl.BlockSpec((_D, _D), lambda i: (0, 0)),
            pl.BlockSpec((_D, _D), lambda i: (0, 0)),
            pl.BlockSpec((1, _D), lambda i: (0, 0)),
        ],
        out_specs=pl.BlockSpec((_FCB, _D), lambda i: (i, 0)),
        out_shape=jax.ShapeDtypeStruct((_B, _D), jnp.float32),
        compiler_params=pltpu.CompilerParams(
            dimension_semantics=("parallel",)),
    )(ge, wt_hi, wt_lo, b2)


def kernel(token_ids, node_emb, edge_weights, edge_matrix, eta, W, b):
    em_flat = edge_matrix.reshape(-1)
    em_pad = jnp.concatenate(
        [em_flat, jnp.zeros((_EPAD - _E,), jnp.int32)])
    em_rows = em_pad.reshape(_NW * _ROWS_PER_W, _CHUNK)
    ew_flat = edge_weights.reshape(-1)
    gathered = _sc_edge_gather(em_rows, ew_flat)
    ew_tab = gathered.reshape(-1)[:_E].reshape(_V, _V)

    tok_pad = jnp.pad(token_ids, ((0, 0), (0, _TP - _T)))
    tok_col = tok_pad.reshape(_B * _TP, 1)
    ne_hi, ne_lo = _split_bf16(node_emb)
    ew_hi = ew_tab.astype(jnp.bfloat16)
    ge = _tc_forward(tok_col, tok_pad, ne_hi, ne_lo, ew_hi,
                     eta.reshape(1, 1))
    wt_hi, wt_lo = _split_bf16(W.T)
    return _fc_forward(ge, wt_hi, wt_lo, b.reshape(1, _D))
